# baseline (speedup 1.0000x reference)
import functools

import jax
import jax.numpy as jnp
from jax import lax
from jax.experimental import pallas as pl
from jax.experimental.pallas import tpu as pltpu
from jax.experimental.pallas import tpu_sc as plsc

NC = 2
NS = 16
LN = 16
C = 128
SUP = 4
D = 32


def _mesh():
    return plsc.VectorSubcoreMesh(core_axis_name="c", subcore_axis_name="s")


@functools.partial(jax.jit, static_argnums=(1,))
def _deg(rows2, nt):
    nch = rows2.shape[0]
    nsup = nch // (NC * NS * SUP)
    degn = ((nt + 1 + NS * 128 - 1) // (NS * 128)) * (NS * 128)
    per_tile = degn // NS

    @functools.partial(
        pl.kernel,
        out_type=jax.ShapeDtypeStruct((NC, 1, degn), jnp.float32),
        mesh=_mesh(),
        scratch_types=[
            pltpu.VMEM_SHARED((degn,), jnp.float32),
            pltpu.VMEM((SUP, C), jnp.int32),
            pltpu.VMEM((C,), jnp.float32),
            pltpu.VMEM((per_tile,), jnp.float32),
            pltpu.SemaphoreType.DMA,
        ],
    )
    def deg_kernel(rows_hbm, out_hbm, acc, idxb, ones, zb, sem):
        cid = lax.axis_index("c")
        sid = lax.axis_index("s")
        w = cid * NS + sid

        def fill_z(i, carry):
            zb[pl.ds(i * LN, LN)] = jnp.zeros((LN,), jnp.float32)
            return carry

        lax.fori_loop(0, per_tile // LN, fill_z, 0)
        for j in range(C // LN):
            ones[pl.ds(j * LN, LN)] = jnp.ones((LN,), jnp.float32)

        pltpu.sync_copy(zb, acc.at[pl.ds(sid * per_tile, per_tile)])
        plsc.subcore_barrier()

        def body(s, carry):
            base = (w * nsup + s) * SUP
            pltpu.sync_copy(rows_hbm.at[pl.ds(base, SUP)], idxb)
            descs = [
                pltpu.async_copy(ones, acc.at[idxb.at[k]], sem, add=True)
                for k in range(SUP)
            ]
            for d in descs:
                d.wait()
            return carry

        lax.fori_loop(0, nsup, body, 0)
        plsc.subcore_barrier()
        pltpu.sync_copy(acc.at[pl.ds(sid * per_tile, per_tile)],
                        out_hbm.at[cid, 0, pl.ds(sid * per_tile, per_tile)])

    return deg_kernel(rows2)


@functools.partial(jax.jit, static_argnums=(3,))
def _agg(g, rows2, cols2, nt):
    nch = rows2.shape[0]
    nsup = nch // (NS * SUP)
    half = nt // 2
    accr = ((half + 1 + NS * 8 - 1) // (NS * 8)) * (NS * 8)
    per_tile = accr // NS
    zr = 64
    nfull, rem = divmod(per_tile, zr)
    wbl = half - (NS - 1) * per_tile

    @functools.partial(
        pl.kernel,
        out_type=jax.ShapeDtypeStruct((nt, D), jnp.float32),
        mesh=_mesh(),
        compiler_params=pltpu.CompilerParams(use_tc_tiling_on_sc=False),
        scratch_types=[
            pltpu.VMEM_SHARED((accr, D), jnp.float32),
            pltpu.VMEM((SUP, C), jnp.int32),
            pltpu.VMEM((SUP, C), jnp.int32),
            pltpu.VMEM((SUP, C), jnp.int32),
            pltpu.VMEM((SUP, C, D), jnp.float32),
            pltpu.VMEM((zr, D), jnp.float32),
            pltpu.SemaphoreType.DMA,
            pltpu.SemaphoreType.DMA,
        ],
    )
    def agg_kernel(g_hbm, rows_hbm, cols_hbm, out_hbm,
                   acc, rb, cb, lb, gb, zb, gsem, ssem):
        cid = lax.axis_index("c")
        sid = lax.axis_index("s")
        base_node = cid * half

        def fill_z(i, carry):
            zb[i, pl.ds(0, LN)] = jnp.zeros((LN,), jnp.float32)
            zb[i, pl.ds(LN, LN)] = jnp.zeros((LN,), jnp.float32)
            return carry

        lax.fori_loop(0, zr, fill_z, 0)
        zoff = sid * per_tile
        for t in range(nfull):
            pltpu.sync_copy(zb, acc.at[pl.ds(zoff + t * zr, zr)])
        if rem:
            pltpu.sync_copy(zb.at[pl.ds(0, rem)],
                            acc.at[pl.ds(zoff + nfull * zr, rem)])
        plsc.subcore_barrier()

        def body(s, carry):
            base = (sid * nsup + s) * SUP
            pltpu.sync_copy(rows_hbm.at[pl.ds(base, SUP)], rb)
            pltpu.sync_copy(cols_hbm.at[pl.ds(base, SUP)], cb)
            gds = []
            for k in range(SUP):
                for j in range(C // LN):
                    rv = rb[k, pl.ds(j * LN, LN)]
                    lv = rv - base_node
                    ok = (lv >= 0) & (lv < half)
                    lb[k, pl.ds(j * LN, LN)] = jnp.where(ok, lv, half)
            sds = []
            for k in range(SUP):
                sds.append(
                    pltpu.async_copy(gb.at[k], acc.at[lb.at[k]], ssem, add=True))
            for d in sds:
                d.wait()
            return carry

        lax.fori_loop(0, nsup, body, 0)
        plsc.subcore_barrier()

        wo = sid * per_tile

        @pl.when(sid < NS - 1)
        def _():
            pltpu.sync_copy(acc.at[pl.ds(wo, per_tile)],
                            out_hbm.at[pl.ds(base_node + wo, per_tile)])

        @pl.when(sid == NS - 1)
        def _():
            pltpu.sync_copy(acc.at[pl.ds((NS - 1) * per_tile, wbl)],
                            out_hbm.at[pl.ds(base_node + (NS - 1) * per_tile, wbl)])

    return agg_kernel(g, rows2, cols2)


_R = 2000


def _row_specs(shapes):
    return [pl.BlockSpec((_R, s), lambda i: (i, 0)) for s in shapes]


def _prep(d0, d1, x):
    nt = x.shape[0]

    def body(d0_ref, d1_ref, x_ref, dinv_ref, g_ref):
        dinv = lax.rsqrt(d0_ref[...] + d1_ref[...] + 1e-8)
        dinv_ref[...] = dinv
        g_ref[...] = x_ref[...] * dinv

    return pl.pallas_call(
        body,
        grid=(nt // _R,),
        in_specs=_row_specs([1, 1, D]),
        out_specs=_row_specs([1, D]),
        out_shape=(jax.ShapeDtypeStruct((nt, 1), jnp.float32),
                   jax.ShapeDtypeStruct((nt, D), jnp.float32)),
    )(d0, d1, x)


def _scale_mid(a, dinv, accp):
    nt = a.shape[0]

    def body(a_ref, d_ref, p_ref, g_ref, acc_ref):
        dv = d_ref[...]
        h = a_ref[...] * dv
        g_ref[...] = h * dv
        acc_ref[...] = p_ref[...] + h

    return pl.pallas_call(
        body,
        grid=(nt // _R,),
        in_specs=_row_specs([D, 1, D]),
        out_specs=_row_specs([D, D]),
        out_shape=(jax.ShapeDtypeStruct((nt, D), jnp.float32),
                   jax.ShapeDtypeStruct((nt, D), jnp.float32)),
    )(a, dinv, accp)


def _scale_final(a, dinv, accp):
    nt = a.shape[0]

    def body(a_ref, d_ref, p_ref, o_ref):
        o_ref[...] = (p_ref[...] + a_ref[...] * d_ref[...]) * 0.25

    return pl.pallas_call(
        body,
        grid=(nt // _R,),
        in_specs=_row_specs([D, 1, D]),
        out_specs=pl.BlockSpec((_R, D), lambda i: (i, 0)),
        out_shape=jax.ShapeDtypeStruct((nt, D), jnp.float32),
    )(a, dinv, accp)


def kernel(user_emb, item_emb, edge_index):
    n_users = user_emb.shape[0]
    nt = n_users + item_emb.shape[0]
    rows = edge_index[0]
    cols = edge_index[1]
    x = jnp.concatenate([user_emb, item_emb], axis=0)

    e = rows.shape[0]
    grain = NC * NS * SUP * C
    ep = ((e + grain - 1) // grain) * grain
    if ep != e:
        rows = jnp.concatenate([rows, jnp.full((ep - e,), nt, jnp.int32)])
        cols = jnp.concatenate([cols, jnp.zeros((ep - e,), jnp.int32)])
    rows2 = rows.reshape(ep // C, C)
    cols2 = cols.reshape(ep // C, C)

    degp = _deg(rows2, nt)
    dinv, g = _prep(degp[0, 0, :nt].reshape(nt, 1), degp[1, 0, :nt].reshape(nt, 1), x)

    acc = x
    for layer in range(3):
        a = _agg(g, rows2, cols2, nt)
        if layer < 2:
            g, acc = _scale_mid(a, dinv, acc)
        else:
            out = _scale_final(a, dinv, acc)
    return out[:n_users], out[n_users:]


# speedup vs baseline: 1.6441x; 1.6441x over previous
import functools

import jax
import jax.numpy as jnp
from jax import lax
from jax.experimental import pallas as pl
from jax.experimental.pallas import tpu as pltpu
from jax.experimental.pallas import tpu_sc as plsc

NC = 2
NS = 16
NW = NC * NS
LN = 16
C = 128
SUP = 4
D = 32

FLUSH = 1024
STG = 2048
CAPC = 408
CAPE = CAPC * C


def _mesh():
    return plsc.VectorSubcoreMesh(core_axis_name="c", subcore_axis_name="s")


@functools.partial(jax.jit, static_argnums=(1,))
def _deg(rows2, nt):
    nch = rows2.shape[0]
    nsup = nch // (NW * SUP)
    degn = ((nt + 1 + NS * 128 - 1) // (NS * 128)) * (NS * 128)
    per_tile = degn // NS

    @functools.partial(
        pl.kernel,
        out_type=jax.ShapeDtypeStruct((NC, 1, degn), jnp.float32),
        mesh=_mesh(),
        scratch_types=[
            pltpu.VMEM_SHARED((degn,), jnp.float32),
            pltpu.VMEM((SUP, C), jnp.int32),
            pltpu.VMEM((C,), jnp.float32),
            pltpu.VMEM((per_tile,), jnp.float32),
            pltpu.SemaphoreType.DMA,
        ],
    )
    def deg_kernel(rows_hbm, out_hbm, acc, idxb, ones, zb, sem):
        cid = lax.axis_index("c")
        sid = lax.axis_index("s")
        w = cid * NS + sid

        def fill_z(i, carry):
            zb[pl.ds(i * LN, LN)] = jnp.zeros((LN,), jnp.float32)
            return carry

        lax.fori_loop(0, per_tile // LN, fill_z, 0)
        for j in range(C // LN):
            ones[pl.ds(j * LN, LN)] = jnp.ones((LN,), jnp.float32)

        pltpu.sync_copy(zb, acc.at[pl.ds(sid * per_tile, per_tile)])
        plsc.subcore_barrier()

        def body(s, carry):
            base = (w * nsup + s) * SUP
            pltpu.sync_copy(rows_hbm.at[pl.ds(base, SUP)], idxb)
            descs = [
                pltpu.async_copy(ones, acc.at[idxb.at[k]], sem, add=True)
                for k in range(SUP)
            ]
            for d in descs:
                d.wait()
            return carry

        lax.fori_loop(0, nsup, body, 0)
        plsc.subcore_barrier()
        pltpu.sync_copy(acc.at[pl.ds(sid * per_tile, per_tile)],
                        out_hbm.at[cid, 0, pl.ds(sid * per_tile, per_tile)])

    return deg_kernel(rows2)


@functools.partial(jax.jit, static_argnums=(2,))
def _bucket(rows2, cols2, nt):
    nch = rows2.shape[0]
    npw = nch // NW
    pg = 8
    npages = npw // pg
    half = nt // 2

    @functools.partial(
        pl.kernel,
        out_type=(jax.ShapeDtypeStruct((NC, NW, 1, CAPE), jnp.int32),
                  jax.ShapeDtypeStruct((NC, NW, 1, CAPE), jnp.int32),
                  jax.ShapeDtypeStruct((NC * NW, 1, LN), jnp.int32)),
        mesh=_mesh(),
        compiler_params=pltpu.CompilerParams(use_tc_tiling_on_sc=False,
                                             needs_layout_passes=False),
        scratch_types=[
            pltpu.VMEM((pg, C), jnp.int32),
            pltpu.VMEM((pg, C), jnp.int32),
            pltpu.VMEM((STG,), jnp.int32),
            pltpu.VMEM((STG,), jnp.int32),
            pltpu.VMEM((STG,), jnp.int32),
            pltpu.VMEM((STG,), jnp.int32),
            pltpu.VMEM((LN,), jnp.int32),
        ],
    )
    def bucket_kernel(rows_hbm, cols_hbm, rb_hbm, cb_hbm, cnt_hbm,
                      pr, pc, sr0, sc0, sr1, sc1, cntb):
        cid = lax.axis_index("c")
        sid = lax.axis_index("s")
        w = cid * NS + sid
        stages = ((sr0, sc0), (sr1, sc1))

        def body(page, carry):
            cur0, nf0, cur1, nf1 = carry
            base = w * npw + page * pg
            pltpu.sync_copy(rows_hbm.at[pl.ds(base, pg)], pr)
            pltpu.sync_copy(cols_hbm.at[pl.ds(base, pg)], pc)
            for k in range(pg):
                for j in range(C // LN):
                    rv = pr[k, pl.ds(j * LN, LN)]
                    cv = pc[k, pl.ds(j * LN, LN)]
                    m0 = rv < half
                    cnt0 = plsc.all_reduce_population_count(m0)[0]
                    m1 = jnp.logical_not(m0)
                    plsc.store_compressed(sr0.at[pl.ds(cur0, LN)], rv, mask=m0)
                    plsc.store_compressed(sc0.at[pl.ds(cur0, LN)], cv, mask=m0)
                    plsc.store_compressed(sr1.at[pl.ds(cur1, LN)], rv, mask=m1)
                    plsc.store_compressed(sc1.at[pl.ds(cur1, LN)], cv, mask=m1)
                    cur0 = cur0 + cnt0
                    cur1 = cur1 + (LN - cnt0)
                curs = [cur0, cur1]
                nfs = [nf0, nf1]
                for p in range(2):
                    sref, cref = stages[p]
                    do_flush = curs[p] >= FLUSH

                    @pl.when(do_flush)
                    def _(p=p, sref=sref, cref=cref, nf=nfs[p]):
                        off = nf * FLUSH
                        pltpu.sync_copy(
                            sref.at[pl.ds(0, FLUSH)],
                            rb_hbm.at[p, w, 0, pl.ds(off, FLUSH)])
                        pltpu.sync_copy(
                            cref.at[pl.ds(0, FLUSH)],
                            cb_hbm.at[p, w, 0, pl.ds(off, FLUSH)])
                        for j in range(C // LN):
                            tr = sref[pl.ds(FLUSH + j * LN, LN)]
                            sref[pl.ds(j * LN, LN)] = tr
                            tc = cref[pl.ds(FLUSH + j * LN, LN)]
                            cref[pl.ds(j * LN, LN)] = tc

                    curs[p] = jnp.where(do_flush, curs[p] - FLUSH, curs[p])
                    nfs[p] = jnp.where(do_flush, nfs[p] + 1, nfs[p])
                cur0, cur1 = curs
                nf0, nf1 = nfs
            return (cur0, nf0, cur1, nf1)

        z = jnp.int32(0)
        cur0, nf0, cur1, nf1 = lax.fori_loop(0, npages, body, (z, z, z, z))

        for p, (cur, nf) in enumerate(((cur0, nf0), (cur1, nf1))):
            sref, cref = stages[p]
            for j in range(512 // LN):
                pos = (jnp.full((LN,), cur, jnp.int32)
                       + lax.iota(jnp.int32, LN) + j * LN)
                plsc.store_scatter(sref, [pos], jnp.full((LN,), nt, jnp.int32))
                plsc.store_scatter(cref, [pos], jnp.zeros((LN,), jnp.int32))
            padded = ((cur + 511) // 512) * 512
            off = nf * FLUSH
            pltpu.sync_copy(sref.at[pl.ds(0, 1536)],
                            rb_hbm.at[p, w, 0, pl.ds(off, 1536)])
            pltpu.sync_copy(cref.at[pl.ds(0, 1536)],
                            cb_hbm.at[p, w, 0, pl.ds(off, 1536)])
            nsup = (off + padded) // (SUP * C)
            cntb[...] = jnp.full((LN,), nsup, jnp.int32)
            pltpu.sync_copy(cntb, cnt_hbm.at[p * NW + w, 0])

    return bucket_kernel(rows2, cols2)


@functools.partial(jax.jit, static_argnums=(4,))
def _agg(g, rows_b, cols_b, counts, nt):
    half = nt // 2
    accr = ((half + 1 + NS * 8 - 1) // (NS * 8)) * (NS * 8)
    per_tile = accr // NS
    zr = 64
    nfull, rem = divmod(per_tile, zr)
    wbl = half - (NS - 1) * per_tile

    @functools.partial(
        pl.kernel,
        out_type=jax.ShapeDtypeStruct((nt, D), jnp.float32),
        mesh=_mesh(),
        compiler_params=pltpu.CompilerParams(use_tc_tiling_on_sc=False,
                                             needs_layout_passes=False),
        scratch_types=[
            pltpu.VMEM_SHARED((accr, D), jnp.float32),
            pltpu.VMEM((SUP * C,), jnp.int32),
            pltpu.VMEM((SUP * C,), jnp.int32),
            pltpu.VMEM((SUP, C), jnp.int32),
            pltpu.VMEM((SUP, C, D), jnp.float32),
            pltpu.VMEM((zr, D), jnp.float32),
            pltpu.VMEM((LN,), jnp.int32),
            pltpu.SemaphoreType.DMA,
            pltpu.SemaphoreType.DMA,
        ],
    )
    def agg_kernel(g_hbm, rows_hbm, cols_hbm, cnt_hbm, out_hbm,
                   acc, rb, cb, lb, gb, zb, cntb, gsem, ssem):
        cid = lax.axis_index("c")
        sid = lax.axis_index("s")
        base_node = cid * half

        def fill_z(i, carry):
            zb[i, pl.ds(0, LN)] = jnp.zeros((LN,), jnp.float32)
            zb[i, pl.ds(LN, LN)] = jnp.zeros((LN,), jnp.float32)
            return carry

        lax.fori_loop(0, zr, fill_z, 0)
        zoff = sid * per_tile
        for t in range(nfull):
            pltpu.sync_copy(zb, acc.at[pl.ds(zoff + t * zr, zr)])
        if rem:
            pltpu.sync_copy(zb.at[pl.ds(0, rem)],
                            acc.at[pl.ds(zoff + nfull * zr, rem)])
        plsc.subcore_barrier()

        for rg in range(2):
            w = sid * 2 + rg
            pltpu.sync_copy(cnt_hbm.at[cid * NW + w, 0], cntb)
            nsup = cntb[pl.ds(0, LN)][0]

            def body(s, carry):
                ebase = s * (SUP * C)
                pltpu.sync_copy(rows_hbm.at[cid, w, 0, pl.ds(ebase, SUP * C)], rb)
                pltpu.sync_copy(cols_hbm.at[cid, w, 0, pl.ds(ebase, SUP * C)], cb)
                gds = [
                    pltpu.async_copy(g_hbm.at[cb.at[pl.ds(k * C, C)]],
                                     gb.at[k], gsem)
                    for k in range(SUP)
                ]
                for k in range(SUP):
                    for j in range(C // LN):
                        rv = rb[pl.ds(k * C + j * LN, LN)]
                        lv = rv - base_node
                        ok = (lv >= 0) & (lv < half)
                        lb[k, pl.ds(j * LN, LN)] = jnp.where(ok, lv, half)
                sds = []
                for k in range(SUP):
                    gds[k].wait()
                    sds.append(
                        pltpu.async_copy(gb.at[k], acc.at[lb.at[k]],
                                         ssem, add=True))
                for d in sds:
                    d.wait()
                return carry

            lax.fori_loop(0, nsup, body, 0)

        plsc.subcore_barrier()

        wo = sid * per_tile

        @pl.when(sid < NS - 1)
        def _():
            pltpu.sync_copy(acc.at[pl.ds(wo, per_tile)],
                            out_hbm.at[pl.ds(base_node + wo, per_tile)])

        @pl.when(sid == NS - 1)
        def _():
            pltpu.sync_copy(acc.at[pl.ds((NS - 1) * per_tile, wbl)],
                            out_hbm.at[pl.ds(base_node + (NS - 1) * per_tile, wbl)])

    return agg_kernel(g, rows_b, cols_b, counts)


_R = 2000


def _row_specs(shapes):
    return [pl.BlockSpec((_R, s), lambda i: (i, 0)) for s in shapes]


def _prep(d0, d1, x):
    nt = x.shape[0]

    def body(d0_ref, d1_ref, x_ref, dinv_ref, g_ref):
        dinv = lax.rsqrt(d0_ref[...] + d1_ref[...] + 1e-8)
        dinv_ref[...] = dinv
        g_ref[...] = x_ref[...] * dinv

    return pl.pallas_call(
        body,
        grid=(nt // _R,),
        in_specs=_row_specs([1, 1, D]),
        out_specs=_row_specs([1, D]),
        out_shape=(jax.ShapeDtypeStruct((nt, 1), jnp.float32),
                   jax.ShapeDtypeStruct((nt, D), jnp.float32)),
    )(d0, d1, x)


def _scale_mid(a, dinv, accp):
    nt = a.shape[0]

    def body(a_ref, d_ref, p_ref, g_ref, acc_ref):
        dv = d_ref[...]
        h = a_ref[...] * dv
        g_ref[...] = h * dv
        acc_ref[...] = p_ref[...] + h

    return pl.pallas_call(
        body,
        grid=(nt // _R,),
        in_specs=_row_specs([D, 1, D]),
        out_specs=_row_specs([D, D]),
        out_shape=(jax.ShapeDtypeStruct((nt, D), jnp.float32),
                   jax.ShapeDtypeStruct((nt, D), jnp.float32)),
    )(a, dinv, accp)


def _scale_final(a, dinv, accp):
    nt = a.shape[0]

    def body(a_ref, d_ref, p_ref, o_ref):
        o_ref[...] = (p_ref[...] + a_ref[...] * d_ref[...]) * 0.25

    return pl.pallas_call(
        body,
        grid=(nt // _R,),
        in_specs=_row_specs([D, 1, D]),
        out_specs=pl.BlockSpec((_R, D), lambda i: (i, 0)),
        out_shape=jax.ShapeDtypeStruct((nt, D), jnp.float32),
    )(a, dinv, accp)


def kernel(user_emb, item_emb, edge_index):
    n_users = user_emb.shape[0]
    nt = n_users + item_emb.shape[0]
    rows = edge_index[0]
    cols = edge_index[1]
    x = jnp.concatenate([user_emb, item_emb], axis=0)

    e = rows.shape[0]
    grain = NW * 8 * C
    ep = ((e + grain - 1) // grain) * grain
    if ep != e:
        rows = jnp.concatenate([rows, jnp.full((ep - e,), nt, jnp.int32)])
        cols = jnp.concatenate([cols, jnp.zeros((ep - e,), jnp.int32)])
    rows2 = rows.reshape(ep // C, C)
    cols2 = cols.reshape(ep // C, C)

    degp = _deg(rows2, nt)
    rows_b, cols_b, counts = _bucket(rows2, cols2, nt)
    dinv, g = _prep(degp[0, 0, :nt].reshape(nt, 1), degp[1, 0, :nt].reshape(nt, 1), x)

    acc = x
    for layer in range(3):
        a = _agg(g, rows_b, cols_b, counts, nt)
        if layer < 2:
            g, acc = _scale_mid(a, dinv, acc)
        else:
            out = _scale_final(a, dinv, acc)
    return out[:n_users], out[n_users:]


# speedup vs baseline: 1.7288x; 1.0515x over previous
import functools

import jax
import jax.numpy as jnp
from jax import lax
from jax.experimental import pallas as pl
from jax.experimental.pallas import tpu as pltpu
from jax.experimental.pallas import tpu_sc as plsc

NC = 2
NS = 16
NW = NC * NS
LN = 16
C = 128
SUP = 4
D = 32

FLUSH = 1024
STG = 2048
CAPC = 408
CAPE = CAPC * C


def _mesh():
    return plsc.VectorSubcoreMesh(core_axis_name="c", subcore_axis_name="s")


@functools.partial(jax.jit, static_argnums=(2,))
def _degbucket(rows2, cols2, nt):
    nch = rows2.shape[0]
    npw = nch // NW
    pg = 8
    npages = npw // pg
    half = nt // 2
    degn = ((nt + 1 + NS * 128 - 1) // (NS * 128)) * (NS * 128)
    per_tile = degn // NS

    @functools.partial(
        pl.kernel,
        out_type=(jax.ShapeDtypeStruct((NC, NW, 1, CAPE), jnp.int32),
                  jax.ShapeDtypeStruct((NC, NW, 1, CAPE), jnp.int32),
                  jax.ShapeDtypeStruct((NC * NW, 1, LN), jnp.int32),
                  jax.ShapeDtypeStruct((NC, 1, degn), jnp.float32)),
        mesh=_mesh(),
        compiler_params=pltpu.CompilerParams(use_tc_tiling_on_sc=False,
                                             needs_layout_passes=False),
        scratch_types=[
            pltpu.VMEM_SHARED((degn,), jnp.float32),
            pltpu.VMEM((pg, C), jnp.int32),
            pltpu.VMEM((pg, C), jnp.int32),
            pltpu.VMEM((STG,), jnp.int32),
            pltpu.VMEM((STG,), jnp.int32),
            pltpu.VMEM((STG,), jnp.int32),
            pltpu.VMEM((STG,), jnp.int32),
            pltpu.VMEM((LN,), jnp.int32),
            pltpu.VMEM((C,), jnp.float32),
            pltpu.VMEM((per_tile,), jnp.float32),
            pltpu.SemaphoreType.DMA,
        ],
    )
    def degbucket_kernel(rows_hbm, cols_hbm, rb_hbm, cb_hbm, cnt_hbm, deg_hbm,
                         dacc, pr, pc, sr0, sc0, sr1, sc1, cntb, ones, zb, sem):
        cid = lax.axis_index("c")
        sid = lax.axis_index("s")
        w = cid * NS + sid
        stages = ((sr0, sc0), (sr1, sc1))

        def fill_z(i, carry):
            zb[pl.ds(i * LN, LN)] = jnp.zeros((LN,), jnp.float32)
            return carry

        lax.fori_loop(0, per_tile // LN, fill_z, 0)
        for j in range(C // LN):
            ones[pl.ds(j * LN, LN)] = jnp.ones((LN,), jnp.float32)
        pltpu.sync_copy(zb, dacc.at[pl.ds(sid * per_tile, per_tile)])
        plsc.subcore_barrier()

        def body(page, carry):
            cur0, nf0, cur1, nf1 = carry
            base = w * npw + page * pg
            pltpu.sync_copy(rows_hbm.at[pl.ds(base, pg)], pr)
            pltpu.sync_copy(cols_hbm.at[pl.ds(base, pg)], pc)
            ddescs = [
                pltpu.async_copy(ones, dacc.at[pr.at[k]], sem, add=True)
                for k in range(pg)
            ]
            for k in range(pg):
                for j in range(C // LN):
                    rv = pr[k, pl.ds(j * LN, LN)]
                    cv = pc[k, pl.ds(j * LN, LN)]
                    m0 = rv < half
                    cnt0 = plsc.all_reduce_population_count(m0)[0]
                    m1 = jnp.logical_not(m0)
                    plsc.store_compressed(sr0.at[pl.ds(cur0, LN)], rv, mask=m0)
                    plsc.store_compressed(sc0.at[pl.ds(cur0, LN)], cv, mask=m0)
                    plsc.store_compressed(sr1.at[pl.ds(cur1, LN)], rv, mask=m1)
                    plsc.store_compressed(sc1.at[pl.ds(cur1, LN)], cv, mask=m1)
                    cur0 = cur0 + cnt0
                    cur1 = cur1 + (LN - cnt0)
                curs = [cur0, cur1]
                nfs = [nf0, nf1]
                for p in range(2):
                    sref, cref = stages[p]
                    do_flush = curs[p] >= FLUSH

                    @pl.when(do_flush)
                    def _(p=p, sref=sref, cref=cref, nf=nfs[p]):
                        off = nf * FLUSH
                        pltpu.sync_copy(
                            sref.at[pl.ds(0, FLUSH)],
                            rb_hbm.at[p, w, 0, pl.ds(off, FLUSH)])
                        pltpu.sync_copy(
                            cref.at[pl.ds(0, FLUSH)],
                            cb_hbm.at[p, w, 0, pl.ds(off, FLUSH)])
                        for j in range(C // LN):
                            tr = sref[pl.ds(FLUSH + j * LN, LN)]
                            sref[pl.ds(j * LN, LN)] = tr
                            tc = cref[pl.ds(FLUSH + j * LN, LN)]
                            cref[pl.ds(j * LN, LN)] = tc

                    curs[p] = jnp.where(do_flush, curs[p] - FLUSH, curs[p])
                    nfs[p] = jnp.where(do_flush, nfs[p] + 1, nfs[p])
                cur0, cur1 = curs
                nf0, nf1 = nfs
            for dd in ddescs:
                dd.wait()
            return (cur0, nf0, cur1, nf1)

        z = jnp.int32(0)
        cur0, nf0, cur1, nf1 = lax.fori_loop(0, npages, body, (z, z, z, z))

        for p, (cur, nf) in enumerate(((cur0, nf0), (cur1, nf1))):
            sref, cref = stages[p]
            for j in range(512 // LN):
                pos = (jnp.full((LN,), cur, jnp.int32)
                       + lax.iota(jnp.int32, LN) + j * LN)
                plsc.store_scatter(sref, [pos], jnp.full((LN,), nt, jnp.int32))
                plsc.store_scatter(cref, [pos], jnp.zeros((LN,), jnp.int32))
            padded = ((cur + 511) // 512) * 512
            off = nf * FLUSH
            pltpu.sync_copy(sref.at[pl.ds(0, 1536)],
                            rb_hbm.at[p, w, 0, pl.ds(off, 1536)])
            pltpu.sync_copy(cref.at[pl.ds(0, 1536)],
                            cb_hbm.at[p, w, 0, pl.ds(off, 1536)])
            nsup = (off + padded) // (SUP * C)
            cntb[...] = jnp.full((LN,), nsup, jnp.int32)
            pltpu.sync_copy(cntb, cnt_hbm.at[p * NW + w, 0])

        plsc.subcore_barrier()
        pltpu.sync_copy(dacc.at[pl.ds(sid * per_tile, per_tile)],
                        deg_hbm.at[cid, 0, pl.ds(sid * per_tile, per_tile)])

    return degbucket_kernel(rows2, cols2)


@functools.partial(jax.jit, static_argnums=(4, 5))
def _agg(g, rows_b, cols_b, counts, nt, mode, dinvE=None, hsum=None, x=None):
    half = nt // 2
    accr = ((half + 1 + NS * 8 - 1) // (NS * 8)) * (NS * 8)
    per_tile = accr // NS
    zr = 64
    nfull, rem = divmod(per_tile, zr)

    un = half // C
    tail = half - un * C
    unq, unr = divmod(un, NS)

    if mode == 2:
        out_type = jax.ShapeDtypeStruct((nt, D), jnp.float32)
    else:
        out_type = (jax.ShapeDtypeStruct((nt, D), jnp.float32),
                    jax.ShapeDtypeStruct((nt, D), jnp.float32))

    @functools.partial(
        pl.kernel,
        out_type=out_type,
        mesh=_mesh(),
        compiler_params=pltpu.CompilerParams(use_tc_tiling_on_sc=False,
                                             needs_layout_passes=False),
        scratch_types=[
            pltpu.VMEM_SHARED((accr, D), jnp.float32),
            pltpu.VMEM((SUP * C,), jnp.int32),
            pltpu.VMEM((SUP * C,), jnp.int32),
            pltpu.VMEM((SUP, C), jnp.int32),
            pltpu.VMEM((SUP, C, D), jnp.float32),
            pltpu.VMEM((zr, D), jnp.float32),
            pltpu.VMEM((LN,), jnp.int32),
            pltpu.SemaphoreType.DMA,
            pltpu.SemaphoreType.DMA,
        ],
    )
    def agg_kernel(*refs):
        if mode == 0:
            (g_hbm, rows_hbm, cols_hbm, cnt_hbm, dE_hbm,
             go_hbm, ho_hbm,
             acc, rb, cb, lb, gb, zb, cntb, gsem, ssem) = refs
            hi_hbm = x_hbm = o_hbm = None
        elif mode == 1:
            (g_hbm, rows_hbm, cols_hbm, cnt_hbm, dE_hbm, hi_hbm,
             go_hbm, ho_hbm,
             acc, rb, cb, lb, gb, zb, cntb, gsem, ssem) = refs
            x_hbm = o_hbm = None
        else:
            (g_hbm, rows_hbm, cols_hbm, cnt_hbm, dE_hbm, hi_hbm, x_hbm,
             o_hbm,
             acc, rb, cb, lb, gb, zb, cntb, gsem, ssem) = refs
            go_hbm = ho_hbm = None

        cid = lax.axis_index("c")
        sid = lax.axis_index("s")
        base_node = cid * half

        def fill_z(i, carry):
            zb[i, pl.ds(0, LN)] = jnp.zeros((LN,), jnp.float32)
            zb[i, pl.ds(LN, LN)] = jnp.zeros((LN,), jnp.float32)
            return carry

        lax.fori_loop(0, zr, fill_z, 0)
        zoff = sid * per_tile
        for t in range(nfull):
            pltpu.sync_copy(zb, acc.at[pl.ds(zoff + t * zr, zr)])
        if rem:
            pltpu.sync_copy(zb.at[pl.ds(0, rem)],
                            acc.at[pl.ds(zoff + nfull * zr, rem)])
        plsc.subcore_barrier()

        for rg in range(2):
            w = sid * 2 + rg
            pltpu.sync_copy(cnt_hbm.at[cid * NW + w, 0], cntb)
            nsup = cntb[pl.ds(0, LN)][0]

            def body(s, carry):
                ebase = s * (SUP * C)
                pltpu.sync_copy(rows_hbm.at[cid, w, 0, pl.ds(ebase, SUP * C)], rb)
                pltpu.sync_copy(cols_hbm.at[cid, w, 0, pl.ds(ebase, SUP * C)], cb)
                gds = [
                    pltpu.async_copy(g_hbm.at[cb.at[pl.ds(k * C, C)]],
                                     gb.at[k], gsem)
                    for k in range(SUP)
                ]
                for k in range(SUP):
                    for j in range(C // LN):
                        rv = rb[pl.ds(k * C + j * LN, LN)]
                        lv = rv - base_node
                        ok = (lv >= 0) & (lv < half)
                        lb[k, pl.ds(j * LN, LN)] = jnp.where(ok, lv, half)
                sds = []
                for k in range(SUP):
                    gds[k].wait()
                    sds.append(
                        pltpu.async_copy(gb.at[k], acc.at[lb.at[k]],
                                         ssem, add=True))
                for d in sds:
                    d.wait()
                return carry

            lax.fori_loop(0, nsup, body, 0)

        plsc.subcore_barrier()

        ab, dEb, hb, ob = gb.at[0], gb.at[1], gb.at[2], gb.at[3]

        def unit_work(u, nrows):
            gofs = base_node + u * C
            pltpu.sync_copy(acc.at[pl.ds(u * C, nrows)],
                            ab.at[pl.ds(0, nrows)])
            pltpu.sync_copy(dE_hbm.at[pl.ds(gofs, nrows)],
                            dEb.at[pl.ds(0, nrows)])
            if mode >= 1:
                pltpu.sync_copy(hi_hbm.at[pl.ds(gofs, nrows)],
                                hb.at[pl.ds(0, nrows)])
            if mode == 2:
                pltpu.sync_copy(x_hbm.at[pl.ds(gofs, nrows)],
                                ob.at[pl.ds(0, nrows)])
            def band(b, carry):
                for q in range(8):
                    r = b * 4 + q // 2
                    csl = pl.ds((q % 2) * LN, LN)
                    a = ab[r, csl]
                    dv = dEb[r, csl]
                    h = a * dv
                    if mode == 0:
                        ob[r, csl] = h * dv
                        hb[r, csl] = h
                    elif mode == 1:
                        ob[r, csl] = h * dv
                        hb[r, csl] = hb[r, csl] + h
                    else:
                        ob[r, csl] = (ob[r, csl] + hb[r, csl] + h) * 0.25
                return carry

            lax.fori_loop(0, nrows // 4, band, 0)
            if mode == 2:
                pltpu.sync_copy(ob.at[pl.ds(0, nrows)],
                                o_hbm.at[pl.ds(gofs, nrows)])
            else:
                pltpu.sync_copy(ob.at[pl.ds(0, nrows)],
                                go_hbm.at[pl.ds(gofs, nrows)])
                pltpu.sync_copy(hb.at[pl.ds(0, nrows)],
                                ho_hbm.at[pl.ds(gofs, nrows)])

        cnt_units = unq + (sid < unr).astype(jnp.int32)

        def ubody(s, carry):
            unit_work(sid + s * NS, C)
            return carry

        lax.fori_loop(0, cnt_units, ubody, 0)

        if tail:
            @pl.when(sid == NS - 1)
            def _():
                unit_work(jnp.int32(un), tail)

    ins = [g, rows_b, cols_b, counts, dinvE]
    if mode >= 1:
        ins.append(hsum)
    if mode == 2:
        ins.append(x)
    return agg_kernel(*ins)


_R = 2000


def _prep(d0, d1, x):
    nt = x.shape[0]

    def body(d0_ref, d1_ref, x_ref, dE_ref, g_ref):
        dinv = lax.rsqrt(d0_ref[...] + d1_ref[...] + 1e-8)
        dE_ref[...] = dinv + jnp.zeros((_R, D), jnp.float32)
        g_ref[...] = x_ref[...] * dinv

    return pl.pallas_call(
        body,
        grid=(nt // _R,),
        in_specs=[pl.BlockSpec((_R, s), lambda i: (i, 0)) for s in (1, 1, D)],
        out_specs=[pl.BlockSpec((_R, s), lambda i: (i, 0)) for s in (D, D)],
        out_shape=(jax.ShapeDtypeStruct((nt, D), jnp.float32),
                   jax.ShapeDtypeStruct((nt, D), jnp.float32)),
    )(d0, d1, x)


def kernel(user_emb, item_emb, edge_index):
    n_users = user_emb.shape[0]
    nt = n_users + item_emb.shape[0]
    rows = edge_index[0]
    cols = edge_index[1]
    x = jnp.concatenate([user_emb, item_emb], axis=0)

    e = rows.shape[0]
    grain = NW * 8 * C
    ep = ((e + grain - 1) // grain) * grain
    if ep != e:
        rows = jnp.concatenate([rows, jnp.full((ep - e,), nt, jnp.int32)])
        cols = jnp.concatenate([cols, jnp.zeros((ep - e,), jnp.int32)])
    rows2 = rows.reshape(ep // C, C)
    cols2 = cols.reshape(ep // C, C)

    rows_b, cols_b, counts, degp = _degbucket(rows2, cols2, nt)
    dinvE, g0 = _prep(degp[0, 0, :nt].reshape(nt, 1),
                      degp[1, 0, :nt].reshape(nt, 1), x)

    g1, hs1 = _agg(g0, rows_b, cols_b, counts, nt, 0, dinvE=dinvE)
    g2, hs2 = _agg(g1, rows_b, cols_b, counts, nt, 1, dinvE=dinvE, hsum=hs1)
    out = _agg(g2, rows_b, cols_b, counts, nt, 2, dinvE=dinvE, hsum=hs2, x=x)
    return out[:n_users], out[n_users:]
